# baseline (speedup 1.0000x reference)
import jax
import jax.numpy as jnp
from jax import lax
from jax.experimental import pallas as pl
from jax.experimental.pallas import tpu as pltpu

_NUM_FED = 50


def _fed_loss_kernel(pred_ref, labels_ref, src_ref, tgt_ref, fedw_ref, gum_ref,
                     out_ref, w_ref, to_ref):
    b = pl.program_id(0)
    Bk, Mk = labels_ref.shape
    Ck = fedw_ref.shape[1]
    Qk = pred_ref.shape[1]

    @pl.when(b == 0)
    def _init():
        labels = labels_ref[...]
        tgt = tgt_ref[...]
        m_iota = lax.broadcasted_iota(jnp.int32, (Bk, Mk, Mk), 2)
        eq3 = tgt[:, :, None] == m_iota
        t_o = jnp.sum(jnp.where(eq3, labels[:, None, :], 0), axis=2)
        to_ref[...] = t_o
        c_iota = lax.broadcasted_iota(jnp.int32, (Bk * Mk, Ck), 1)
        hits = (t_o.reshape(Bk * Mk, 1) == c_iota).astype(jnp.float32)
        uniq = jnp.max(hits, axis=0, keepdims=True)
        n_u = jnp.sum(uniq).astype(jnp.int32)
        prob = fedw_ref[...] * (1.0 - uniq)
        g = -gum_ref[...] - jnp.log(prob)
        g_col = g.reshape(Ck, 1)
        j_lt_c = (lax.broadcasted_iota(jnp.int32, (Ck, Ck), 0)
                  < lax.broadcasted_iota(jnp.int32, (Ck, Ck), 1))
        before = (g_col < g) | ((g_col == g) & j_lt_c)
        rank = jnp.sum(before.astype(jnp.int32), axis=0, keepdims=True)
        extra = (rank < (_NUM_FED - n_u)).astype(jnp.float32)
        w_ref[...] = jnp.maximum(uniq, extra)

    x = pred_ref[0]
    sp = jnp.maximum(x, 0.0) + jnp.log1p(jnp.exp(-jnp.abs(x)))
    src_b = src_ref[pl.ds(b, 1), :]
    t_ob = to_ref[pl.ds(b, 1), :]
    q_col = lax.broadcasted_iota(jnp.int32, (Qk, 1), 0)
    match = q_col == src_b
    t_q = jnp.sum(jnp.where(match, t_ob - Ck, 0), axis=1,
                  keepdims=True) + Ck
    c_row = lax.broadcasted_iota(jnp.int32, (1, Ck), 1)
    eqm = c_row == t_q
    contrib = jnp.sum(w_ref[...] * sp - jnp.where(eqm, x, 0.0),
                      keepdims=True).reshape(1, 1)
    out_ref[...] = jnp.where(b == 0, contrib, out_ref[...] + contrib)


def kernel(pred_logits, fed_loss_cls_weights, labels, src_idx, tgt_idx,
           num_boxes):
    B_, Q_, C_ = pred_logits.shape
    M_ = labels.shape[1]
    gum = jax.random.gumbel(jax.random.key(42), (C_ + 1,), jnp.float32)[:C_]
    gum = gum.reshape(1, C_)
    fedw = fed_loss_cls_weights.astype(jnp.float32).reshape(1, C_)
    labels_i = labels.astype(jnp.int32)
    src_i = src_idx.astype(jnp.int32)
    tgt_i = tgt_idx.astype(jnp.int32)

    out = pl.pallas_call(
        _fed_loss_kernel,
        grid=(B_,),
        in_specs=[
            pl.BlockSpec((1, Q_, C_), lambda b: (b, 0, 0)),
            pl.BlockSpec((B_, M_), lambda b: (0, 0)),
            pl.BlockSpec((B_, M_), lambda b: (0, 0)),
            pl.BlockSpec((B_, M_), lambda b: (0, 0)),
            pl.BlockSpec((1, C_), lambda b: (0, 0)),
            pl.BlockSpec((1, C_), lambda b: (0, 0)),
        ],
        out_specs=pl.BlockSpec((1, 1), lambda b: (0, 0)),
        out_shape=jax.ShapeDtypeStruct((1, 1), jnp.float32),
        scratch_shapes=[
            pltpu.VMEM((1, C_), jnp.float32),
            pltpu.VMEM((B_, M_), jnp.int32),
        ],
    )(pred_logits, labels_i, src_i, tgt_i, fedw, gum)
    return out[0, 0] / (B_ * Q_)


# speedup vs baseline: 1.0765x; 1.0765x over previous
import jax
import jax.numpy as jnp
from jax import lax
from jax.experimental import pallas as pl
from jax.experimental.pallas import tpu as pltpu

_NUM_FED = 50


def _prep_kernel(labels_ref, tgt_ref, fedw_ref, gum_ref, w_ref, to_ref):
    Bk, Mk = labels_ref.shape
    Ck = fedw_ref.shape[1]
    labels = labels_ref[...]
    tgt = tgt_ref[...]
    m_iota = lax.broadcasted_iota(jnp.int32, (Bk, Mk, Mk), 2)
    eq3 = tgt[:, :, None] == m_iota
    t_o = jnp.sum(jnp.where(eq3, labels[:, None, :], 0), axis=2)
    to_ref[...] = t_o
    c_iota = lax.broadcasted_iota(jnp.int32, (Bk * Mk, Ck), 1)
    hits = (t_o.reshape(Bk * Mk, 1) == c_iota).astype(jnp.float32)
    uniq = jnp.max(hits, axis=0, keepdims=True)
    n_u = jnp.sum(uniq).astype(jnp.int32)
    prob = fedw_ref[...] * (1.0 - uniq)
    g = -gum_ref[...] - jnp.log(prob)
    g_col = g.reshape(Ck, 1)
    j_lt_c = (lax.broadcasted_iota(jnp.int32, (Ck, Ck), 0)
              < lax.broadcasted_iota(jnp.int32, (Ck, Ck), 1))
    before = (g_col < g) | ((g_col == g) & j_lt_c)
    rank = jnp.sum(before.astype(jnp.int32), axis=0, keepdims=True)
    extra = (rank < (_NUM_FED - n_u)).astype(jnp.float32)
    w_ref[...] = jnp.maximum(uniq, extra)


def _dense_kernel(pred_ref, src_ref, to_ref, w_ref, out_ref):
    b = pl.program_id(0)
    Bk, Mk = src_ref.shape
    Ck = w_ref.shape[1]
    Qk = pred_ref.shape[1]
    x = pred_ref[0]
    sp = jnp.maximum(x, 0.0) + jnp.log1p(jnp.exp(-jnp.abs(x)))
    cols = jnp.sum(sp, axis=0, keepdims=True)
    term = jnp.sum(cols * w_ref[...], keepdims=True).reshape(1, 1)
    src_col = src_ref[pl.ds(b, 1), :].reshape(Mk, 1)
    t_col = to_ref[pl.ds(b, 1), :].reshape(Mk, 1)
    q_row = lax.broadcasted_iota(jnp.int32, (1, Qk), 1)
    oh = (src_col == q_row).astype(jnp.float32)
    rows = lax.dot_general(oh, x, (((1,), (0,)), ((), ())),
                           preferred_element_type=jnp.float32)
    c_row = lax.broadcasted_iota(jnp.int32, (1, Ck), 1)
    corr = jnp.sum(jnp.where(t_col == c_row, rows, 0.0),
                   keepdims=True).reshape(1, 1)
    out_ref[...] = (term - corr).reshape(1, 1, 1)


def kernel(pred_logits, fed_loss_cls_weights, labels, src_idx, tgt_idx,
           num_boxes):
    B_, Q_, C_ = pred_logits.shape
    M_ = labels.shape[1]
    gum = jax.random.gumbel(jax.random.key(42), (C_ + 1,), jnp.float32)[:C_]
    gum = gum.reshape(1, C_)
    fedw = fed_loss_cls_weights.astype(jnp.float32).reshape(1, C_)
    labels_i = labels.astype(jnp.int32)
    src_i = src_idx.astype(jnp.int32)
    tgt_i = tgt_idx.astype(jnp.int32)

    w, t_o = pl.pallas_call(
        _prep_kernel,
        out_shape=(
            jax.ShapeDtypeStruct((1, C_), jnp.float32),
            jax.ShapeDtypeStruct((B_, M_), jnp.int32),
        ),
    )(labels_i, tgt_i, fedw, gum)

    partials = pl.pallas_call(
        _dense_kernel,
        grid=(B_,),
        in_specs=[
            pl.BlockSpec((1, Q_, C_), lambda b: (b, 0, 0)),
            pl.BlockSpec((B_, M_), lambda b: (0, 0)),
            pl.BlockSpec((B_, M_), lambda b: (0, 0)),
            pl.BlockSpec((1, C_), lambda b: (0, 0)),
        ],
        out_specs=pl.BlockSpec((1, 1, 1), lambda b: (b, 0, 0)),
        out_shape=jax.ShapeDtypeStruct((B_, 1, 1), jnp.float32),
        compiler_params=pltpu.CompilerParams(
            dimension_semantics=("parallel",)),
    )(pred_logits, src_i, t_o, w)
    return jnp.sum(partials) / (B_ * Q_)


# speedup vs baseline: 1.1896x; 1.1051x over previous
import jax
import jax.numpy as jnp
from jax import lax
from jax.experimental import pallas as pl
from jax.experimental.pallas import tpu as pltpu

_NUM_FED = 50
_LANES = 128


def _prep_kernel(labels_ref, tgt_ref, fedw_ref, gum_ref, w_ref, to_ref):
    Bk, Mk = labels_ref.shape
    Cp = fedw_ref.shape[1]
    labels = labels_ref[...]
    tgt = tgt_ref[...]
    m_iota = lax.broadcasted_iota(jnp.int32, (Bk, Mk, Mk), 2)
    eq3 = tgt[:, :, None] == m_iota
    t_o = jnp.sum(jnp.where(eq3, labels[:, None, :], 0), axis=2)
    to_ref[...] = t_o
    c_iota = lax.broadcasted_iota(jnp.int32, (Bk * Mk, Cp), 1)
    hits = (t_o.reshape(Bk * Mk, 1) == c_iota).astype(jnp.float32)
    uniq = jnp.max(hits, axis=0, keepdims=True)
    n_u = jnp.sum(uniq).astype(jnp.int32)
    prob = fedw_ref[...] * (1.0 - uniq)
    g = -gum_ref[...] - jnp.log(prob)
    g_col = g.reshape(Cp, 1)
    j_lt_c = (lax.broadcasted_iota(jnp.int32, (Cp, Cp), 0)
              < lax.broadcasted_iota(jnp.int32, (Cp, Cp), 1))
    before = (g_col < g) | ((g_col == g) & j_lt_c)
    rank = jnp.sum(before.astype(jnp.int32), axis=0, keepdims=True)
    extra = (rank < (_NUM_FED - n_u)).astype(jnp.float32)
    w_ref[...] = jnp.maximum(uniq, extra)


def _make_dense_kernel(c_valid):
    def _dense_kernel(pa_ref, pb_ref, src_ref, to_ref, w_ref, out_ref):
        b = pl.program_id(0)
        Bk, Mk = src_ref.shape
        nimg = pa_ref.shape[0]
        Qk = pa_ref.shape[1]
        Wa = pa_ref.shape[2]
        Wb = pb_ref.shape[2]
        q_row = lax.broadcasted_iota(jnp.int32, (1, Qk), 1)
        ca_row = lax.broadcasted_iota(jnp.int32, (1, Wa), 1)
        cb_row = lax.broadcasted_iota(jnp.int32, (1, Wb), 1)
        acc = jnp.zeros((1, 1), jnp.float32)
        for i in range(nimg):
            xa = pa_ref[i]
            xb = pb_ref[i]
            spa = jnp.maximum(xa, 0.0) + jnp.log(1.0 + jnp.exp(-jnp.abs(xa)))
            spb = jnp.maximum(xb, 0.0) + jnp.log(1.0 + jnp.exp(-jnp.abs(xb)))
            cols_a = jnp.sum(spa, axis=0, keepdims=True)
            cols_b = jnp.sum(spb, axis=0, keepdims=True)
            cols_b = jnp.where(cb_row < c_valid - Wa, cols_b, 0.0)
            term = (jnp.sum(cols_a * w_ref[:, :Wa], keepdims=True)
                    + jnp.sum(cols_b * w_ref[:, Wa:],
                              keepdims=True)).reshape(1, 1)
            img = b * nimg + i
            src_col = src_ref[pl.ds(img, 1), :].reshape(Mk, 1)
            t_col = to_ref[pl.ds(img, 1), :].reshape(Mk, 1)
            oh = (src_col == q_row).astype(jnp.float32)
            rows_a = lax.dot_general(oh, xa, (((1,), (0,)), ((), ())),
                                     preferred_element_type=jnp.float32)
            rows_b = lax.dot_general(oh, xb, (((1,), (0,)), ((), ())),
                                     preferred_element_type=jnp.float32)
            corr = (jnp.sum(jnp.where(t_col == ca_row, rows_a, 0.0),
                            keepdims=True)
                    + jnp.sum(jnp.where(t_col - Wa == cb_row, rows_b, 0.0),
                              keepdims=True)).reshape(1, 1)
            acc = acc + term - corr
        out_ref[...] = acc.reshape(1, 1, 1)
    return _dense_kernel


def kernel(pred_logits, fed_loss_cls_weights, labels, src_idx, tgt_idx,
           num_boxes):
    B_, Q_, C_ = pred_logits.shape
    M_ = labels.shape[1]
    Cp = ((C_ + _LANES - 1) // _LANES) * _LANES
    if Cp % 256:
        Cp += _LANES
    gum = jax.random.gumbel(jax.random.key(42), (C_ + 1,), jnp.float32)[:C_]
    gum = jnp.pad(gum, (0, Cp - C_)).reshape(1, Cp)
    fedw = jnp.pad(fed_loss_cls_weights.astype(jnp.float32),
                   (0, Cp - C_)).reshape(1, Cp)
    labels_i = labels.astype(jnp.int32)
    src_i = src_idx.astype(jnp.int32)
    tgt_i = tgt_idx.astype(jnp.int32)

    w, t_o = pl.pallas_call(
        _prep_kernel,
        out_shape=(
            jax.ShapeDtypeStruct((1, Cp), jnp.float32),
            jax.ShapeDtypeStruct((B_, M_), jnp.int32),
        ),
    )(labels_i, tgt_i, fedw, gum)

    nimg = 2
    half = Cp // 2
    partials = pl.pallas_call(
        _make_dense_kernel(C_),
        grid=(B_ // nimg,),
        in_specs=[
            pl.BlockSpec((nimg, Q_, half), lambda b: (b, 0, 0)),
            pl.BlockSpec((nimg, Q_, half), lambda b: (b, 0, 1)),
            pl.BlockSpec((B_, M_), lambda b: (0, 0)),
            pl.BlockSpec((B_, M_), lambda b: (0, 0)),
            pl.BlockSpec((1, Cp), lambda b: (0, 0)),
        ],
        out_specs=pl.BlockSpec((1, 1, 1), lambda b: (b, 0, 0)),
        out_shape=jax.ShapeDtypeStruct((B_ // nimg, 1, 1), jnp.float32),
        compiler_params=pltpu.CompilerParams(
            dimension_semantics=("parallel",)),
    )(pred_logits, pred_logits, src_i, t_o, w)
    return jnp.sum(partials) / (B_ * Q_)


# speedup vs baseline: 1.2369x; 1.0398x over previous
import jax
import jax.numpy as jnp
from jax import lax
from jax.experimental import pallas as pl
from jax.experimental.pallas import tpu as pltpu

_NUM_FED = 50
_LANES = 128


def _prep_kernel(labels_ref, tgt_ref, fedw_ref, gum_ref, w_ref, to_ref):
    Bk, Mk = labels_ref.shape
    Cp = fedw_ref.shape[1]
    labels = labels_ref[...]
    tgt = tgt_ref[...]
    m_iota = lax.broadcasted_iota(jnp.int32, (Bk, Mk, Mk), 2)
    eq3 = tgt[:, :, None] == m_iota
    t_o = jnp.sum(jnp.where(eq3, labels[:, None, :], 0), axis=2)
    to_ref[...] = t_o
    c_iota = lax.broadcasted_iota(jnp.int32, (Bk * Mk, Cp), 1)
    hits = (t_o.reshape(Bk * Mk, 1) == c_iota).astype(jnp.float32)
    uniq = jnp.max(hits, axis=0, keepdims=True)
    n_u = jnp.sum(uniq).astype(jnp.int32)
    prob = fedw_ref[...] * (1.0 - uniq)
    g = -gum_ref[...] - jnp.log(prob)
    g_col = g.reshape(Cp, 1)
    j_lt_c = (lax.broadcasted_iota(jnp.int32, (Cp, Cp), 0)
              < lax.broadcasted_iota(jnp.int32, (Cp, Cp), 1))
    before = (g_col < g) | ((g_col == g) & j_lt_c)
    rank = jnp.sum(before.astype(jnp.int32), axis=0, keepdims=True)
    extra = (rank < (_NUM_FED - n_u)).astype(jnp.float32)
    w_ref[...] = jnp.maximum(uniq, extra)


def _make_dense_kernel(q_valid):
    def _dense_kernel(pa_ref, pb_ref, src_ref, to_ref, w_ref, out_ref):
        b = pl.program_id(0)
        Bk, Mk = src_ref.shape
        nimg = pa_ref.shape[0]
        Qh = pa_ref.shape[1]
        Ck = pa_ref.shape[2]
        n_pad = 2 * Qh - q_valid
        q_row_a = lax.broadcasted_iota(jnp.int32, (1, Qh), 1)
        c_row = lax.broadcasted_iota(jnp.int32, (1, Ck), 1)
        row_col = lax.broadcasted_iota(jnp.int32, (Qh, 1), 0)
        ln2 = jnp.float32(0.6931471805599453)
        acc = jnp.zeros((1, 1), jnp.float32)
        for i in range(nimg):
            xa = pa_ref[i]
            xb = jnp.where(row_col < q_valid - Qh, pb_ref[i], 0.0)
            spa = jnp.maximum(xa, 0.0) + jnp.log(1.0 + jnp.exp(-jnp.abs(xa)))
            spb = jnp.maximum(xb, 0.0) + jnp.log(1.0 + jnp.exp(-jnp.abs(xb)))
            cols = (jnp.sum(spa, axis=0, keepdims=True)
                    + jnp.sum(spb, axis=0, keepdims=True))
            w_row = w_ref[...]
            term = (jnp.sum(cols * w_row, keepdims=True)
                    - ln2 * n_pad * jnp.sum(w_row,
                                            keepdims=True)).reshape(1, 1)
            img = b * nimg + i
            src_col = src_ref[pl.ds(img, 1), :].reshape(Mk, 1)
            t_col = to_ref[pl.ds(img, 1), :].reshape(Mk, 1)
            oh_a = (src_col == q_row_a).astype(jnp.float32)
            oh_b = (src_col - Qh == q_row_a).astype(jnp.float32)
            rows = (lax.dot_general(oh_a, xa, (((1,), (0,)), ((), ())),
                                    preferred_element_type=jnp.float32)
                    + lax.dot_general(oh_b, xb, (((1,), (0,)), ((), ())),
                                      preferred_element_type=jnp.float32))
            corr = jnp.sum(jnp.where(t_col == c_row, rows, 0.0),
                           keepdims=True).reshape(1, 1)
            acc = acc + term - corr
        out_ref[...] = acc.reshape(1, 1, 1)
    return _dense_kernel


def kernel(pred_logits, fed_loss_cls_weights, labels, src_idx, tgt_idx,
           num_boxes):
    B_, Q_, C_ = pred_logits.shape
    M_ = labels.shape[1]
    gum = jax.random.gumbel(jax.random.key(42), (C_ + 1,), jnp.float32)[:C_]
    gum = gum.reshape(1, C_)
    fedw = fed_loss_cls_weights.astype(jnp.float32).reshape(1, C_)
    labels_i = labels.astype(jnp.int32)
    src_i = src_idx.astype(jnp.int32)
    tgt_i = tgt_idx.astype(jnp.int32)

    w, t_o = pl.pallas_call(
        _prep_kernel,
        out_shape=(
            jax.ShapeDtypeStruct((1, C_), jnp.float32),
            jax.ShapeDtypeStruct((B_, M_), jnp.int32),
        ),
    )(labels_i, tgt_i, fedw, gum)

    nimg = 2
    q_half = ((Q_ // 2 + 7) // 8) * 8
    if 2 * q_half < Q_:
        q_half += 8
    partials = pl.pallas_call(
        _make_dense_kernel(Q_),
        grid=(B_ // nimg,),
        in_specs=[
            pl.BlockSpec((nimg, q_half, C_), lambda b: (b, 0, 0)),
            pl.BlockSpec((nimg, q_half, C_), lambda b: (b, 1, 0)),
            pl.BlockSpec((B_, M_), lambda b: (0, 0)),
            pl.BlockSpec((B_, M_), lambda b: (0, 0)),
            pl.BlockSpec((1, C_), lambda b: (0, 0)),
        ],
        out_specs=pl.BlockSpec((1, 1, 1), lambda b: (b, 0, 0)),
        out_shape=jax.ShapeDtypeStruct((B_ // nimg, 1, 1), jnp.float32),
        compiler_params=pltpu.CompilerParams(
            dimension_semantics=("parallel",)),
    )(pred_logits, pred_logits, src_i, t_o, w)
    return jnp.sum(partials) / (B_ * Q_)
